# baseline (speedup 1.0000x reference)
import functools

import jax
import jax.numpy as jnp
import numpy as np
from jax import lax
from jax.experimental import pallas as pl
from jax.experimental.pallas import tpu as pltpu
from jax.experimental.pallas import tpu_sc as plsc

LOG2PI = float(np.log(2.0 * np.pi))
MAX_LOGIT = 20.0
B = 16
R = 32768
E = 64
K = 16

NUM_WORKERS = 32
CHUNK = R // NUM_WORKERS
ROW_W = 24
F = 8
TC_BLOCK = 512


def _log_erfc(z):
    az = jnp.abs(z)
    t = 1.0 / (1.0 + 0.5 * az)
    p = t * (1.00002368 + t * (0.37409196 + t * (0.09678418 + t * (
        -0.18628806 + t * (0.27886807 + t * (-1.13520398 + t * (
            1.48851587 + t * (-0.82215223 + t * 0.17087277)))))))) - 1.26551223
    q = p - z * z
    pos = z >= 0.0
    val = jnp.where(pos, t, 2.0 - t * jnp.exp(q))
    return jnp.log(val) + jnp.where(pos, q, 0.0)


def _tile_f(v):
    return jnp.concatenate([v] * F, axis=1)


def _dense_body(x_ref, p_ref, dirs_ref, out_ref, w_s, g_s, t_s, s_s, n_s, p_s):
    dn = (((1,), (1,)), ((), ()))

    @pl.when(pl.program_id(0) == 0)
    def _build_constants():
        s_e = p_ref[0:1, :]
        asig = _tile_f(p_ref[1:2, 0:K])
        mu = _tile_f(p_ref[2:3, 0:K])
        sig = _tile_f(p_ref[3:4, 0:K])
        lam = _tile_f(p_ref[4:5, 0:K])

        dirs = dirs_ref[...]
        unit = dirs * lax.rsqrt(jnp.sum(dirs * dirs, axis=-1, keepdims=True))

        inv_s = 1.0 / s_e
        c_na = -(E / 2.0) * LOG2PI - jnp.sum(jnp.log(s_e), axis=-1,
                                             keepdims=True)
        c_out = c_na - E * float(np.log(2.0))
        c_orth = (-((E - 1) / 2.0) * LOG2PI - (E - 1) * jnp.log(asig))
        inv2sig2 = 1.0 / (2.0 * asig * asig)
        a_k = mu + lam * sig * sig
        c_par = jnp.log(0.5 * lam) - 0.5 * (lam * sig) * (lam * sig)
        inv_sqrt2sig = 1.0 / (float(np.sqrt(2.0)) * sig)

        zke = jnp.zeros((K, E), dtype=jnp.float32)
        wrows = []
        for j in range(F):
            wrows.append(jnp.concatenate(
                [zke] * j + [unit] + [zke] * (F - 1 - j), axis=1))
        w_s[...] = jnp.concatenate(wrows, axis=0)

        gr = lax.broadcasted_iota(jnp.int32, (2 * F, F * E), 0)
        gc = lax.broadcasted_iota(jnp.int32, (2 * F, F * E), 1) // E
        gind_lo = jnp.where(gr == gc, 1.0, 0.0)
        gind_hi = jnp.where(gr == gc + F, 1.0, 0.0)
        g_s[...] = gind_lo + gind_hi * _tile_f(inv_s * inv_s)
        tr = lax.broadcasted_iota(jnp.int32, (F * K, F), 0) // K
        tcc = lax.broadcasted_iota(jnp.int32, (F * K, F), 1)
        t_s[...] = jnp.where(tr == tcc, 1.0, 0.0) * inv2sig2.reshape(F * K, 1)

        ci = lax.broadcasted_iota(jnp.int32, (F * ROW_W, F * K), 0)
        mi = lax.broadcasted_iota(jnp.int32, (F * ROW_W, F * K), 1)
        s_s[...] = jnp.where(
            ci == ROW_W * (mi // K) + (mi - K * (mi // K)), 1.0, 0.0)
        ci2 = lax.broadcasted_iota(jnp.int32, (F * ROW_W, 2 * F), 0)
        ji2 = lax.broadcasted_iota(jnp.int32, (F * ROW_W, 2 * F), 1)
        tgt = jnp.where(ji2 < F, ROW_W * ji2 + K, ROW_W * (ji2 - F) + K + 1)
        n_s[...] = jnp.where(ci2 == tgt, 1.0, 0.0)

        p_s[0:1, :] = _tile_f(inv_s)
        zpad = jnp.zeros((1, F * E - F * K), dtype=jnp.float32)
        p_s[1:2, :] = jnp.concatenate([c_orth + c_par, zpad], axis=1)
        p_s[2:3, :] = jnp.concatenate([inv2sig2, zpad], axis=1)
        p_s[3:4, :] = jnp.concatenate([a_k, zpad], axis=1)
        p_s[4:5, :] = jnp.concatenate([inv_sqrt2sig, zpad], axis=1)
        p_s[5:6, :] = jnp.concatenate([lam, zpad], axis=1)
        p_s[6:7, :] = jnp.concatenate(
            [c_na, c_out, jnp.zeros((1, F * E - 2), dtype=jnp.float32)],
            axis=1)

    xf = x_ref[...]
    c_art = p_s[1:2, 0:F * K]
    inv2sig2 = p_s[2:3, 0:F * K]
    a_k = p_s[3:4, 0:F * K]
    inv_sqrt2sig = p_s[4:5, 0:F * K]
    lam = p_s[5:6, 0:F * K]
    c_na = p_s[6:7, 0:1]
    c_out = p_s[6:7, 1:2]

    sq = xf * xf
    sw = lax.dot_general(sq, g_s[...], dn,
                         preferred_element_type=jnp.float32)
    s2_8 = sw[:, 0:F]
    w2_8 = sw[:, F:2 * F]
    s2t = lax.dot_general(s2_8, t_s[...], dn,
                          preferred_element_type=jnp.float32)
    dot_f = lax.dot_general(xf, w_s[...], dn,
                            preferred_element_type=jnp.float32)

    na_8 = c_na - 0.5 * w2_8
    ou_8 = c_out - 0.125 * w2_8
    d = a_k - dot_f
    z = d * inv_sqrt2sig
    art_f = (c_art + lam * d + _log_erfc(z)
             + dot_f * dot_f * inv2sig2 - s2t)

    eno = jnp.concatenate([na_8, ou_8], axis=-1)
    out_ref[...] = (
        lax.dot_general(art_f, s_s[...], dn,
                        preferred_element_type=jnp.float32)
        + lax.dot_general(eno, n_s[...], dn,
                          preferred_element_type=jnp.float32))


NSUB = 4
SUB = CHUNK // NSUB
NBANK = 4
BANK_W = 2 * B * K


def _segsum_body(rows_hbm, seg_hbm, out_hbm, buf_a, buf_b, seg_v, acc_v,
                 sem_a, sem_b, sem_s):
    wid = lax.axis_index("s") * 2 + lax.axis_index("c")
    base = wid * CHUNK

    seg_cp = pltpu.async_copy(seg_hbm.at[pl.ds(base, CHUNK)], seg_v, sem_s)
    bufs = (buf_a, buf_b)
    sems = (sem_a, sem_b)

    def start(t):
        return pltpu.async_copy(
            rows_hbm.at[pl.ds((base + t * SUB) * ROW_W, SUB * ROW_W)],
            bufs[t % 2].at[pl.ds(0, SUB * ROW_W)], sems[t % 2])

    cp = start(0)

    zero16 = jnp.zeros((16,), dtype=jnp.float32)
    for i in range(NBANK * BANK_W // 16):
        acc_v[pl.ds(16 * i, 16)] = zero16

    col = lax.iota(jnp.int32, 16)
    colbank = [col + m * BANK_W for m in range(NBANK)]
    ext_mask = col < 2

    seg_cp.wait()
    for t in range(NSUB):
        cp.wait()
        if t + 1 < NSUB:
            cp = start(t + 1)
        buf = bufs[t % 2]

        def group(g, carry):
            sv = seg_v[pl.ds(t * SUB + g * 16, 16)]
            row0 = g * 16
            for j in range(16):
                idx = sv[j] * 16 + colbank[j % NBANK]
                art = buf[pl.ds((row0 + j) * ROW_W, 16)]
                ext = buf[pl.ds((row0 + j) * ROW_W + 16, 16)]
                plsc.addupdate_scatter(acc_v, [idx], art)
                plsc.addupdate_scatter(acc_v, [idx + (16 * B)], ext,
                                       mask=ext_mask)
            return carry

        lax.fori_loop(0, SUB // 16, group, 0)

    pltpu.sync_copy(acc_v, out_hbm.at[wid])


def _finalize_body(p_ref, parts_ref, logits_ref, loglks_ref):
    s = parts_ref[0]
    for i in range(1, NUM_WORKERS * NBANK):
        s = s + parts_ref[i]
    art_bk = s[0:B, :]
    na_b = s[B:2 * B, 0:1]
    ou_b = s[B:2 * B, 1:2]

    cw = p_ref[5:6, 0:K]
    m = jnp.max(cw, axis=-1, keepdims=True)
    log_w = cw - (m + jnp.log(jnp.sum(jnp.exp(cw - m), axis=-1, keepdims=True)))
    art_w = art_bk + log_w

    ma = jnp.max(art_w, axis=-1, keepdims=True)
    alk = ma + jnp.log(jnp.sum(jnp.exp(art_w - ma), axis=-1, keepdims=True))
    logits = alk - na_b
    logits_ref[...] = MAX_LOGIT * jnp.tanh(logits / MAX_LOGIT)
    loglks_ref[...] = jnp.concatenate([na_b, ou_b, art_w], axis=-1)


def kernel(features, segment_ids, nonartifact_stdev_e, artifact_directions_ke,
           artifact_stdev_k, cluster_weights_pre_softmax_k, emg_mu_k,
           emg_sigma_k, emg_rate_k):
    p = jnp.zeros((8, E), dtype=jnp.float32)
    p = p.at[0, :].set(nonartifact_stdev_e)
    p = p.at[1, 0:K].set(artifact_stdev_k)
    p = p.at[2, 0:K].set(emg_mu_k)
    p = p.at[3, 0:K].set(emg_sigma_k)
    p = p.at[4, 0:K].set(emg_rate_k)
    p = p.at[5, 0:K].set(cluster_weights_pre_softmax_k)

    packed = pl.pallas_call(
        _dense_body,
        grid=(R // (F * TC_BLOCK),),
        in_specs=[
            pl.BlockSpec((TC_BLOCK, F * E), lambda i: (i, 0)),
            pl.BlockSpec((8, E), lambda i: (0, 0)),
            pl.BlockSpec((K, E), lambda i: (0, 0)),
        ],
        out_specs=pl.BlockSpec((TC_BLOCK, F * ROW_W), lambda i: (i, 0)),
        out_shape=jax.ShapeDtypeStruct((R // F, F * ROW_W), jnp.float32),
        scratch_shapes=[
            pltpu.VMEM((F * K, F * E), jnp.float32),
            pltpu.VMEM((2 * F, F * E), jnp.float32),
            pltpu.VMEM((F * K, F), jnp.float32),
            pltpu.VMEM((F * ROW_W, F * K), jnp.float32),
            pltpu.VMEM((F * ROW_W, 2 * F), jnp.float32),
            pltpu.VMEM((8, F * E), jnp.float32),
        ],
    )(features.reshape(R // F, F * E), p, artifact_directions_ke)

    segsum = pl.kernel(
        _segsum_body,
        out_type=jax.ShapeDtypeStruct((NUM_WORKERS, NBANK * BANK_W),
                                      jnp.float32),
        mesh=plsc.VectorSubcoreMesh(core_axis_name="c", subcore_axis_name="s",
                                    num_cores=2, num_subcores=16),
        scratch_types=[
            pltpu.VMEM((SUB * ROW_W + 16,), jnp.float32),
            pltpu.VMEM((SUB * ROW_W + 16,), jnp.float32),
            pltpu.VMEM((CHUNK,), jnp.int32),
            pltpu.VMEM((NBANK * BANK_W,), jnp.float32),
            pltpu.SemaphoreType.DMA,
            pltpu.SemaphoreType.DMA,
            pltpu.SemaphoreType.DMA,
        ],
        compiler_params=pltpu.CompilerParams(needs_layout_passes=False),
    )
    partials = segsum(packed.reshape(-1), segment_ids)

    capped, log_lks = pl.pallas_call(
        _finalize_body,
        grid=(1,),
        in_specs=[
            pl.BlockSpec((8, E), lambda i: (0, 0)),
            pl.BlockSpec((NUM_WORKERS * NBANK, 2 * B, K), lambda i: (0, 0, 0)),
        ],
        out_specs=[
            pl.BlockSpec((B, 1), lambda i: (0, 0)),
            pl.BlockSpec((B, K + 2), lambda i: (0, 0)),
        ],
        out_shape=[
            jax.ShapeDtypeStruct((B, 1), jnp.float32),
            jax.ShapeDtypeStruct((B, K + 2), jnp.float32),
        ],
    )(p, partials.reshape(NUM_WORKERS * NBANK, 2 * B, K))

    return capped.reshape(B), log_lks


# speedup vs baseline: 1.2029x; 1.2029x over previous
import functools

import jax
import jax.numpy as jnp
import numpy as np
from jax import lax
from jax.experimental import pallas as pl
from jax.experimental.pallas import tpu as pltpu
from jax.experimental.pallas import tpu_sc as plsc

LOG2PI = float(np.log(2.0 * np.pi))
MAX_LOGIT = 20.0
B = 16
R = 32768
E = 64
K = 16

NUM_WORKERS = 32
CHUNK = R // NUM_WORKERS
F = 8
TC_BLOCK = 512


def _log_erfc(z):
    az = jnp.abs(z)
    t = 1.0 / (1.0 + 0.5 * az)
    p = t * (1.00002368 + t * (0.37409196 + t * (0.09678418 + t * (
        -0.18628806 + t * (0.27886807 + t * (-1.13520398 + t * (
            1.48851587 + t * (-0.82215223 + t * 0.17087277)))))))) - 1.26551223
    q = p - z * z
    pos = z >= 0.0
    val = jnp.where(pos, t, 2.0 - t * jnp.exp(q))
    return jnp.log(val) + jnp.where(pos, q, 0.0)


def _tile_f(v):
    return jnp.concatenate([v] * F, axis=1)


def _dense_body(x_ref, p_ref, dirs_ref, out_ref, no_ref, w_s, g_s, t_s, y_s,
                p_s):
    dn = (((1,), (1,)), ((), ()))

    @pl.when(pl.program_id(0) == 0)
    def _build_constants():
        s_e = p_ref[0:1, :]
        asig = _tile_f(p_ref[1:2, 0:K])
        mu = _tile_f(p_ref[2:3, 0:K])
        sig = _tile_f(p_ref[3:4, 0:K])
        lam = _tile_f(p_ref[4:5, 0:K])

        dirs = dirs_ref[...]
        unit = dirs * lax.rsqrt(jnp.sum(dirs * dirs, axis=-1, keepdims=True))

        inv_s = 1.0 / s_e
        c_na = -(E / 2.0) * LOG2PI - jnp.sum(jnp.log(s_e), axis=-1,
                                             keepdims=True)
        c_out = c_na - E * float(np.log(2.0))
        c_orth = (-((E - 1) / 2.0) * LOG2PI - (E - 1) * jnp.log(asig))
        inv2sig2 = 1.0 / (2.0 * asig * asig)
        a_k = mu + lam * sig * sig
        c_par = jnp.log(0.5 * lam) - 0.5 * (lam * sig) * (lam * sig)
        inv_sqrt2sig = 1.0 / (float(np.sqrt(2.0)) * sig)

        zke = jnp.zeros((K, E), dtype=jnp.float32)
        wrows = []
        for j in range(F):
            wrows.append(jnp.concatenate(
                [zke] * j + [unit] + [zke] * (F - 1 - j), axis=1))
        w_s[...] = jnp.concatenate(wrows, axis=0)
        gr = lax.broadcasted_iota(jnp.int32, (F * F, F * E), 0)
        gc = lax.broadcasted_iota(jnp.int32, (F * F, F * E), 1) // E
        same_j = (gr // F) == gc
        g_s[...] = (jnp.where(jnp.logical_and(same_j, (gr % F) == 0), 1.0, 0.0)
                    * _tile_f(inv_s * inv_s)
                    + jnp.where(jnp.logical_and(same_j, (gr % F) == 1),
                                1.0, 0.0))
        tr = lax.broadcasted_iota(jnp.int32, (F * K, E), 0)
        tm = lax.broadcasted_iota(jnp.int32, (F * K, E), 1)
        t_s[...] = (jnp.where(tm == F * (tr // K) + 1, 1.0, 0.0)
                    * inv2sig2.reshape(F * K, 1))
        y_s[...] = jnp.where(
            jnp.logical_and(tr < F, tm == F * tr), -0.5, 0.0) + jnp.where(
            jnp.logical_and(jnp.logical_and(tr >= F, tr < 2 * F),
                            tm == F * (tr - F)), -0.125, 0.0)

        p_s[0:1, :] = _tile_f(inv_s)
        zpad = jnp.zeros((1, F * E - F * K), dtype=jnp.float32)
        p_s[1:2, :] = jnp.concatenate([c_orth + c_par, zpad], axis=1)
        p_s[2:3, :] = jnp.concatenate([inv2sig2, zpad], axis=1)
        p_s[3:4, :] = jnp.concatenate([a_k, zpad], axis=1)
        p_s[4:5, :] = jnp.concatenate([inv_sqrt2sig, zpad], axis=1)
        p_s[5:6, :] = jnp.concatenate([lam, zpad], axis=1)
        cl = lax.broadcasted_iota(jnp.int32, (1, F * E), 1)
        p_s[6:7, :] = (jnp.where(cl < F, c_na, 0.0)
                       + jnp.where(jnp.logical_and(cl >= F, cl < 2 * F),
                                   c_out, 0.0))

    xf = x_ref[...].reshape(TC_BLOCK, F * E)
    c_art = p_s[1:2, 0:F * K]
    inv2sig2 = p_s[2:3, 0:F * K]
    a_k = p_s[3:4, 0:F * K]
    inv_sqrt2sig = p_s[4:5, 0:F * K]
    lam = p_s[5:6, 0:F * K]
    cvec = p_s[6:7, 0:F * K]

    dot_f = lax.dot_general(xf, w_s[...], dn,
                            preferred_element_type=jnp.float32)
    wsf = lax.dot_general(xf * xf, g_s[...], dn,
                          preferred_element_type=jnp.float32)
    s2t = lax.dot_general(wsf, t_s[...], dn,
                          preferred_element_type=jnp.float32)
    no_lin = lax.dot_general(wsf, y_s[...], dn,
                             preferred_element_type=jnp.float32)

    d = a_k - dot_f
    z = d * inv_sqrt2sig
    art_f = (c_art + lam * d + _log_erfc(z)
             + dot_f * dot_f * inv2sig2 - s2t)

    out_ref[...] = art_f
    no_ref[...] = no_lin + cvec


NSUB = 4
SUB = CHUNK // NSUB
NBANK = 4
BANK_W = 2 * B * K

_GATHER_DN = lax.GatherDimensionNumbers(
    offset_dims=(), collapsed_slice_dims=(0,), start_index_map=(0,))


def _gather16(vec, idx):
    return lax.gather(vec, idx[:, None], _GATHER_DN, slice_sizes=(1,),
                      mode=lax.GatherScatterMode.PROMISE_IN_BOUNDS)


def _segsum_body(art_hbm, no_hbm, seg_hbm, out_hbm, abuf_a, abuf_b,
                 nbuf_a, nbuf_b, seg_v, acc_v, sem_a, sem_b, sem_s):
    wid = lax.axis_index("s") * 2 + lax.axis_index("c")
    base = wid * CHUNK

    seg_cp = pltpu.async_copy(seg_hbm.at[pl.ds(base, CHUNK)], seg_v, sem_s)
    abufs = (abuf_a, abuf_b)
    nbufs = (nbuf_a, nbuf_b)
    sems = (sem_a, sem_b)

    def start(t):
        a = pltpu.async_copy(
            art_hbm.at[pl.ds((base + t * SUB) * K, SUB * K)],
            abufs[t % 2], sems[t % 2])
        n = pltpu.async_copy(
            no_hbm.at[pl.ds((base + t * SUB) * K, SUB * K)],
            nbufs[t % 2], sems[t % 2])
        return a, n

    cp = start(0)

    zero16 = jnp.zeros((16,), dtype=jnp.float32)
    for i in range(NBANK * BANK_W // (F * K)):
        for l in range(F):
            acc_v[i, pl.ds(16 * l, 16)] = zero16

    col = lax.iota(jnp.int32, 16)
    colbank = [col + m * BANK_W for m in range(NBANK)]
    perm_a = jnp.bitwise_and(col, 7)
    perm_b = perm_a + 8

    seg_cp.wait()
    for t in range(NSUB):
        cp[0].wait()
        cp[1].wait()
        if t + 1 < NSUB:
            cp = start(t + 1)
        abuf = abufs[t % 2]
        nbuf = nbufs[t % 2]

        def group(g, carry):
            sv = seg_v[pl.ds(t * SUB + g * 16, 16)]
            row0 = g * 16
            noa = nbuf[pl.ds((g * 2) * F * K, 16)]
            nob = nbuf[pl.ds((g * 2 + 1) * F * K, 16)]
            s0 = sv[0]
            eq = sv == jnp.full((16,), s0, dtype=jnp.int32)
            nsame = plsc.all_reduce_population_count(eq)
            boff = jnp.bitwise_and(2 * g, NBANK - 1) * BANK_W

            def scat(flat_idx, val):
                plsc.addupdate_scatter(
                    acc_v, [lax.shift_right_logical(flat_idx, 7),
                            jnp.bitwise_and(flat_idx, 127)], val)

            def fast():
                vals = [abuf[pl.ds((row0 + j) * K, 16)] for j in range(16)]
                while len(vals) > 1:
                    vals = [vals[i] + vals[i + 1]
                            for i in range(0, len(vals), 2)]
                scat(s0 * 16 + col + boff, vals[0])
                scat((s0 + B) * 16 + col + (boff + BANK_W), noa + nob)

            def slow():
                for j in range(16):
                    idx = sv[j] * 16 + colbank[j % NBANK]
                    art = abuf[pl.ds((row0 + j) * K, 16)]
                    scat(idx, art)
                sda = _gather16(sv, perm_a)
                sdb = _gather16(sv, perm_b)
                scat((sda + B) * 16 + col + boff, noa)
                scat((sdb + B) * 16 + col + (boff + BANK_W), nob)

            lax.cond(nsame[0] == 16, fast, slow)
            return carry

        lax.fori_loop(0, SUB // 16, group, 0)

    nrow = NBANK * BANK_W // (F * K)
    pltpu.sync_copy(acc_v, out_hbm.at[pl.ds(wid * nrow, nrow), :])


def _finalize_body(p_ref, parts_ref, logits_ref, loglks_ref):
    s4 = parts_ref[0:4, :]
    for i in range(1, NUM_WORKERS * NBANK):
        s4 = s4 + parts_ref[4 * i:4 * i + 4, :]
    s = jnp.concatenate(
        [s4[b // F:b // F + 1, (b % F) * K:(b % F) * K + K]
         for b in range(2 * B)], axis=0)
    art_bk = s[0:B, :]
    na_b = jnp.sum(s[B:2 * B, 0:F], axis=-1, keepdims=True)
    ou_b = jnp.sum(s[B:2 * B, F:2 * F], axis=-1, keepdims=True)

    cw = p_ref[5:6, 0:K]
    m = jnp.max(cw, axis=-1, keepdims=True)
    log_w = cw - (m + jnp.log(jnp.sum(jnp.exp(cw - m), axis=-1, keepdims=True)))
    art_w = art_bk + log_w

    ma = jnp.max(art_w, axis=-1, keepdims=True)
    alk = ma + jnp.log(jnp.sum(jnp.exp(art_w - ma), axis=-1, keepdims=True))
    logits = alk - na_b
    logits_ref[...] = MAX_LOGIT * jnp.tanh(logits / MAX_LOGIT)
    loglks_ref[...] = jnp.concatenate([na_b, ou_b, art_w], axis=-1)


def kernel(features, segment_ids, nonartifact_stdev_e, artifact_directions_ke,
           artifact_stdev_k, cluster_weights_pre_softmax_k, emg_mu_k,
           emg_sigma_k, emg_rate_k):
    p = jnp.zeros((8, E), dtype=jnp.float32)
    p = p.at[0, :].set(nonartifact_stdev_e)
    p = p.at[1, 0:K].set(artifact_stdev_k)
    p = p.at[2, 0:K].set(emg_mu_k)
    p = p.at[3, 0:K].set(emg_sigma_k)
    p = p.at[4, 0:K].set(emg_rate_k)
    p = p.at[5, 0:K].set(cluster_weights_pre_softmax_k)

    art_p, no_p = pl.pallas_call(
        _dense_body,
        grid=(R // (F * TC_BLOCK),),
        in_specs=[
            pl.BlockSpec((F * TC_BLOCK // 2, 2 * E), lambda i: (i, 0)),
            pl.BlockSpec((8, E), lambda i: (0, 0)),
            pl.BlockSpec((K, E), lambda i: (0, 0)),
        ],
        out_specs=[
            pl.BlockSpec((TC_BLOCK, F * K), lambda i: (i, 0)),
            pl.BlockSpec((TC_BLOCK, F * K), lambda i: (i, 0)),
        ],
        out_shape=[
            jax.ShapeDtypeStruct((R // F, F * K), jnp.float32),
            jax.ShapeDtypeStruct((R // F, F * K), jnp.float32),
        ],
        scratch_shapes=[
            pltpu.VMEM((F * K, F * E), jnp.float32),
            pltpu.VMEM((F * F, F * E), jnp.float32),
            pltpu.VMEM((F * K, E), jnp.float32),
            pltpu.VMEM((F * K, E), jnp.float32),
            pltpu.VMEM((8, F * E), jnp.float32),
        ],
    )(features.reshape(R // 2, 2 * E), p, artifact_directions_ke)

    segsum = pl.kernel(
        _segsum_body,
        out_type=jax.ShapeDtypeStruct(
            (NUM_WORKERS * NBANK * BANK_W // (F * K), F * K), jnp.float32),
        mesh=plsc.VectorSubcoreMesh(core_axis_name="c", subcore_axis_name="s",
                                    num_cores=2, num_subcores=16),
        scratch_types=[
            pltpu.VMEM((SUB * K,), jnp.float32),
            pltpu.VMEM((SUB * K,), jnp.float32),
            pltpu.VMEM((SUB * K,), jnp.float32),
            pltpu.VMEM((SUB * K,), jnp.float32),
            pltpu.VMEM((CHUNK,), jnp.int32),
            pltpu.VMEM((NBANK * BANK_W // (F * K), F * K), jnp.float32),
            pltpu.SemaphoreType.DMA,
            pltpu.SemaphoreType.DMA,
            pltpu.SemaphoreType.DMA,
        ],
        compiler_params=pltpu.CompilerParams(needs_layout_passes=False),
    )
    partials = segsum(art_p.reshape(-1), no_p.reshape(-1), segment_ids)

    capped, log_lks = pl.pallas_call(
        _finalize_body,
        grid=(1,),
        in_specs=[
            pl.BlockSpec((8, E), lambda i: (0, 0)),
            pl.BlockSpec((NUM_WORKERS * NBANK * 4, F * K), lambda i: (0, 0)),
        ],
        out_specs=[
            pl.BlockSpec((B, 1), lambda i: (0, 0)),
            pl.BlockSpec((B, K + 2), lambda i: (0, 0)),
        ],
        out_shape=[
            jax.ShapeDtypeStruct((B, 1), jnp.float32),
            jax.ShapeDtypeStruct((B, K + 2), jnp.float32),
        ],
    )(p, partials)

    return capped.reshape(B), log_lks


# speedup vs baseline: 1.4063x; 1.1691x over previous
import jax
import jax.numpy as jnp
import numpy as np
from jax import lax
from jax.experimental import pallas as pl
from jax.experimental.pallas import tpu as pltpu
from jax.experimental.pallas import tpu_sc as plsc

LOG2PI = float(np.log(2.0 * np.pi))
MAX_LOGIT = 20.0
B = 16
R = 32768
E = 64
K = 16

NUM_WORKERS = 32
CHUNK = R // NUM_WORKERS
F = 8
TC_BLOCK = 1024


def _log_erfc(z):
    az = jnp.abs(z)
    t = 1.0 / (1.0 + 0.5 * az)
    p = t * (1.00002368 + t * (0.37409196 + t * (0.09678418 + t * (
        -0.18628806 + t * (0.27886807 + t * (-1.13520398 + t * (
            1.48851587 + t * (-0.82215223 + t * 0.17087277)))))))) - 1.26551223
    q = p - z * z
    pos = z >= 0.0
    val = jnp.where(pos, t, 2.0 - t * jnp.exp(q))
    return jnp.log(val) + jnp.where(pos, q, 0.0)


def _tile_f(v):
    return jnp.concatenate([v] * F, axis=1)


def _dense_body(x_ref, p_ref, dirs_ref, out_ref, no_ref, w_s, g_s, t_s, y_s,
                p_s):
    dn = (((1,), (1,)), ((), ()))

    @pl.when(pl.program_id(0) == 0)
    def _build_constants():
        s_e = p_ref[0:1, :]
        asig = _tile_f(p_ref[1:2, 0:K])
        mu = _tile_f(p_ref[2:3, 0:K])
        sig = _tile_f(p_ref[3:4, 0:K])
        lam = _tile_f(p_ref[4:5, 0:K])

        dirs = dirs_ref[...]
        unit = dirs * lax.rsqrt(jnp.sum(dirs * dirs, axis=-1, keepdims=True))

        inv_s = 1.0 / s_e
        c_na = -(E / 2.0) * LOG2PI - jnp.sum(jnp.log(s_e), axis=-1,
                                             keepdims=True)
        c_out = c_na - E * float(np.log(2.0))
        c_orth = (-((E - 1) / 2.0) * LOG2PI - (E - 1) * jnp.log(asig))
        inv2sig2 = 1.0 / (2.0 * asig * asig)
        a_k = mu + lam * sig * sig
        c_par = jnp.log(0.5 * lam) - 0.5 * (lam * sig) * (lam * sig)
        inv_sqrt2sig = 1.0 / (float(np.sqrt(2.0)) * sig)

        zk2 = jnp.zeros((K, 2 * E), dtype=jnp.float32)
        up = jnp.concatenate([unit, jnp.zeros((K, E), dtype=jnp.float32)],
                             axis=1)
        wrows = []
        for j in range(F):
            wrows.append(jnp.concatenate(
                [zk2] * j + [up] + [zk2] * (F - 1 - j), axis=1))
        w_s[...] = jnp.concatenate(wrows, axis=0)
        gr = lax.broadcasted_iota(jnp.int32, (F * F, F * 2 * E), 0)
        gc = lax.broadcasted_iota(jnp.int32, (F * F, F * 2 * E), 1)
        same_j = (gr // F) == (gc // (2 * E))
        lane_e = gc - (gc // (2 * E)) * (2 * E)
        invs2_p = jnp.concatenate(
            [inv_s * inv_s, jnp.zeros((1, E), dtype=jnp.float32)], axis=1)
        invs2_big = jnp.concatenate([invs2_p] * F, axis=1)
        g_s[...] = (jnp.where(jnp.logical_and(same_j, (gr % F) == 0), 1.0, 0.0)
                    * invs2_big
                    + jnp.where(jnp.logical_and(
                        jnp.logical_and(same_j, (gr % F) == 1),
                        lane_e < E), 1.0, 0.0))
        tr = lax.broadcasted_iota(jnp.int32, (F * K, E), 0)
        tm = lax.broadcasted_iota(jnp.int32, (F * K, E), 1)
        t_s[...] = (jnp.where(tm == F * (tr // K) + 1, 1.0, 0.0)
                    * inv2sig2.reshape(F * K, 1))
        y_s[...] = jnp.where(
            jnp.logical_and(tr < F, tm == F * tr), -0.5, 0.0) + jnp.where(
            jnp.logical_and(jnp.logical_and(tr >= F, tr < 2 * F),
                            tm == F * (tr - F)), -0.125, 0.0)

        p_s[0:1, :] = _tile_f(inv_s)
        zpad = jnp.zeros((1, F * E - F * K), dtype=jnp.float32)
        p_s[1:2, :] = jnp.concatenate([c_orth + c_par, zpad], axis=1)
        p_s[2:3, :] = jnp.concatenate([inv2sig2, zpad], axis=1)
        p_s[3:4, :] = jnp.concatenate([a_k, zpad], axis=1)
        p_s[4:5, :] = jnp.concatenate([inv_sqrt2sig, zpad], axis=1)
        p_s[5:6, :] = jnp.concatenate([lam, zpad], axis=1)
        cl = lax.broadcasted_iota(jnp.int32, (1, F * E), 1)
        p_s[6:7, :] = (jnp.where(cl < F, c_na, 0.0)
                       + jnp.where(jnp.logical_and(cl >= F, cl < 2 * F),
                                   c_out, 0.0))

    xb = x_ref[...]
    xp = jnp.concatenate(
        [xb, jnp.zeros((F * TC_BLOCK, E), dtype=jnp.float32)], axis=1)
    xf = xp.reshape(TC_BLOCK, F * 2 * E)
    c_art = p_s[1:2, 0:F * K]
    inv2sig2 = p_s[2:3, 0:F * K]
    a_k = p_s[3:4, 0:F * K]
    inv_sqrt2sig = p_s[4:5, 0:F * K]
    lam = p_s[5:6, 0:F * K]
    cvec = p_s[6:7, 0:F * K]

    dot_f = lax.dot_general(xf, w_s[...], dn,
                            preferred_element_type=jnp.float32)
    wsf = lax.dot_general(xf * xf, g_s[...], dn,
                          preferred_element_type=jnp.float32)
    s2t = lax.dot_general(wsf, t_s[...], dn,
                          preferred_element_type=jnp.float32)
    no_lin = lax.dot_general(wsf, y_s[...], dn,
                             preferred_element_type=jnp.float32)

    d = a_k - dot_f
    z = d * inv_sqrt2sig
    art_f = (c_art + lam * d + _log_erfc(z)
             + dot_f * dot_f * inv2sig2 - s2t)

    out_ref[...] = art_f
    no_ref[...] = no_lin + cvec


NSUB = 2
SUB = CHUNK // NSUB
NBANK = 4
BANK_W = 2 * B * K

_GATHER_DN = lax.GatherDimensionNumbers(
    offset_dims=(), collapsed_slice_dims=(0,), start_index_map=(0,))


def _gather16(vec, idx):
    return lax.gather(vec, idx[:, None], _GATHER_DN, slice_sizes=(1,),
                      mode=lax.GatherScatterMode.PROMISE_IN_BOUNDS)


def _make_segsum_body(seg0, chunk, nsub):
    sub = chunk // nsub

    def _segsum_body(art_hbm, no_hbm, seg_hbm, out_hbm, abuf_a, abuf_b,
                     nbuf_a, nbuf_b, seg_v, acc_v, sem_a, sem_b, sem_s):
        wid = lax.axis_index("s") * 2 + lax.axis_index("c")
        base = wid * chunk

        seg_cp = pltpu.async_copy(seg_hbm.at[pl.ds(seg0 + base, chunk)],
                                  seg_v, sem_s)
        abufs = (abuf_a, abuf_b)
        nbufs = (nbuf_a, nbuf_b)
        sems = (sem_a, sem_b)

        def start(t):
            a = pltpu.async_copy(
                art_hbm.at[pl.ds((base + t * sub) * K, sub * K)],
                abufs[t % 2], sems[t % 2])
            n = pltpu.async_copy(
                no_hbm.at[pl.ds((base + t * sub) * K, sub * K)],
                nbufs[t % 2], sems[t % 2])
            return a, n

        cp = start(0)

        zero16 = jnp.zeros((16,), dtype=jnp.float32)
        for i in range(NBANK * BANK_W // (F * K)):
            for l in range(F):
                acc_v[i, pl.ds(16 * l, 16)] = zero16

        col = lax.iota(jnp.int32, 16)
        colbank = [col + m * BANK_W for m in range(NBANK)]
        perm_a = jnp.bitwise_and(col, 7)
        perm_b = perm_a + 8

        seg_cp.wait()
        for t in range(nsub):
            cp[0].wait()
            cp[1].wait()
            if t + 1 < nsub:
                cp = start(t + 1)
            abuf = abufs[t % 2]
            nbuf = nbufs[t % 2]

            def group(g, carry):
                sv = seg_v[pl.ds(t * sub + g * 16, 16)]
                row0 = g * 16
                noa = nbuf[pl.ds((g * 2) * F * K, 16)]
                nob = nbuf[pl.ds((g * 2 + 1) * F * K, 16)]
                s0 = sv[0]
                eq = sv == jnp.full((16,), s0, dtype=jnp.int32)
                nsame = plsc.all_reduce_population_count(eq)
                boff = jnp.bitwise_and(2 * g, NBANK - 1) * BANK_W

                def scat(flat_idx, val):
                    plsc.addupdate_scatter(
                        acc_v, [lax.shift_right_logical(flat_idx, 7),
                                jnp.bitwise_and(flat_idx, 127)], val)

                def fast():
                    vals = [abuf[pl.ds((row0 + j) * K, 16)]
                            for j in range(16)]
                    while len(vals) > 1:
                        vals = [vals[i] + vals[i + 1]
                                for i in range(0, len(vals), 2)]
                    scat(s0 * 16 + col + boff, vals[0])
                    scat((s0 + B) * 16 + col + (boff + BANK_W), noa + nob)

                def slow():
                    for j in range(16):
                        idx = sv[j] * 16 + colbank[j % NBANK]
                        art = abuf[pl.ds((row0 + j) * K, 16)]
                        scat(idx, art)
                    sda = _gather16(sv, perm_a)
                    sdb = _gather16(sv, perm_b)
                    scat((sda + B) * 16 + col + boff, noa)
                    scat((sdb + B) * 16 + col + (boff + BANK_W), nob)

                lax.cond(nsame[0] == 16, fast, slow)
                return carry

            lax.fori_loop(0, sub // 16, group, 0)

        nrow = NBANK * BANK_W // (F * K)
        pltpu.sync_copy(acc_v, out_hbm.at[pl.ds(wid * nrow, nrow), :])

    return _segsum_body


def _finalize_body(p_ref, parts_ref, logits_ref, loglks_ref):
    s4 = parts_ref[0:4, :]
    for i in range(1, NUM_WORKERS * NBANK):
        s4 = s4 + parts_ref[4 * i:4 * i + 4, :]
    s = jnp.concatenate(
        [s4[b // F:b // F + 1, (b % F) * K:(b % F) * K + K]
         for b in range(2 * B)], axis=0)
    art_bk = s[0:B, :]
    na_b = jnp.sum(s[B:2 * B, 0:F], axis=-1, keepdims=True)
    ou_b = jnp.sum(s[B:2 * B, F:2 * F], axis=-1, keepdims=True)

    cw = p_ref[5:6, 0:K]
    m = jnp.max(cw, axis=-1, keepdims=True)
    log_w = cw - (m + jnp.log(jnp.sum(jnp.exp(cw - m), axis=-1, keepdims=True)))
    art_w = art_bk + log_w

    ma = jnp.max(art_w, axis=-1, keepdims=True)
    alk = ma + jnp.log(jnp.sum(jnp.exp(art_w - ma), axis=-1, keepdims=True))
    logits = alk - na_b
    capped = MAX_LOGIT * jnp.tanh(logits / MAX_LOGIT)
    logits_ref[...] = lax.transpose(capped, (1, 0))
    loglks_ref[...] = jnp.concatenate([na_b, ou_b, art_w], axis=-1)


def kernel(features, segment_ids, nonartifact_stdev_e, artifact_directions_ke,
           artifact_stdev_k, cluster_weights_pre_softmax_k, emg_mu_k,
           emg_sigma_k, emg_rate_k):
    zk = jnp.zeros((E - K,), dtype=jnp.float32)
    ze = jnp.zeros((E,), dtype=jnp.float32)
    p = jnp.stack([
        nonartifact_stdev_e,
        jnp.concatenate([artifact_stdev_k, zk]),
        jnp.concatenate([emg_mu_k, zk]),
        jnp.concatenate([emg_sigma_k, zk]),
        jnp.concatenate([emg_rate_k, zk]),
        jnp.concatenate([cluster_weights_pre_softmax_k, zk]),
        ze, ze])

    art_p, no_p = pl.pallas_call(
        _dense_body,
        grid=(R // (F * TC_BLOCK),),
        in_specs=[
            pl.BlockSpec((F * TC_BLOCK, E), lambda i: (i, 0)),
            pl.BlockSpec((8, E), lambda i: (0, 0)),
            pl.BlockSpec((K, E), lambda i: (0, 0)),
        ],
        out_specs=[
            pl.BlockSpec((TC_BLOCK, F * K), lambda i: (i, 0)),
            pl.BlockSpec((TC_BLOCK, F * K), lambda i: (i, 0)),
        ],
        out_shape=[
            jax.ShapeDtypeStruct((R // F, F * K), jnp.float32),
            jax.ShapeDtypeStruct((R // F, F * K), jnp.float32),
        ],
        scratch_shapes=[
            pltpu.VMEM((F * K, F * 2 * E), jnp.float32),
            pltpu.VMEM((F * F, F * 2 * E), jnp.float32),
            pltpu.VMEM((F * K, E), jnp.float32),
            pltpu.VMEM((F * K, E), jnp.float32),
            pltpu.VMEM((8, F * E), jnp.float32),
        ],
    )(features, p, artifact_directions_ke)

    segsum = pl.kernel(
        _make_segsum_body(0, CHUNK, NSUB),
        out_type=jax.ShapeDtypeStruct(
            (NUM_WORKERS * NBANK * BANK_W // (F * K), F * K), jnp.float32),
        mesh=plsc.VectorSubcoreMesh(core_axis_name="c", subcore_axis_name="s",
                                    num_cores=2, num_subcores=16),
        scratch_types=[
            pltpu.VMEM((CHUNK // NSUB * K,), jnp.float32),
            pltpu.VMEM((CHUNK // NSUB * K,), jnp.float32),
            pltpu.VMEM((CHUNK // NSUB * K,), jnp.float32),
            pltpu.VMEM((CHUNK // NSUB * K,), jnp.float32),
            pltpu.VMEM((CHUNK,), jnp.int32),
            pltpu.VMEM((NBANK * BANK_W // (F * K), F * K), jnp.float32),
            pltpu.SemaphoreType.DMA,
            pltpu.SemaphoreType.DMA,
            pltpu.SemaphoreType.DMA,
        ],
        compiler_params=pltpu.CompilerParams(needs_layout_passes=False),
    )
    partials = segsum(art_p.reshape(-1), no_p.reshape(-1), segment_ids)

    capped, log_lks = pl.pallas_call(
        _finalize_body,
        grid=(1,),
        in_specs=[
            pl.BlockSpec((8, E), lambda i: (0, 0)),
            pl.BlockSpec((NUM_WORKERS * NBANK * 4, F * K), lambda i: (0, 0)),
        ],
        out_specs=[
            pl.BlockSpec((1, B), lambda i: (0, 0)),
            pl.BlockSpec((B, K + 2), lambda i: (0, 0)),
        ],
        out_shape=[
            jax.ShapeDtypeStruct((1, B), jnp.float32),
            jax.ShapeDtypeStruct((B, K + 2), jnp.float32),
        ],
    )(p, partials)

    return capped.reshape(B), log_lks
